# baseline (speedup 1.0000x reference)
import functools
import numpy as np
import jax
import jax.numpy as jnp
from jax import lax
from jax.experimental import pallas as pl
from jax.experimental.pallas import tpu as pltpu
from jax.experimental.pallas import tpu_sc as plsc

D = 768
MAX_LEN = 512
NLANE = 16
NSLICE = D // NLANE
C = 32
NW = 32


def _pe_table():
    position = np.arange(0, MAX_LEN, dtype=np.float32)[:, None]
    div_term = np.exp(
        np.arange(0, D, 2, dtype=np.float32) * -(np.log(10000.0) / D)
    )
    pe = np.zeros((MAX_LEN, D), dtype=np.float32)
    pe[:, 0::2] = np.sin(position * div_term)
    pe[:, 1::2] = np.cos(position * div_term)
    return pe


@functools.lru_cache(maxsize=None)
def _make_kernel(B, L, interpret=False):
    TOK = B * L
    rows_per_w = B // NW
    n_lc = L // C
    mesh = plsc.VectorSubcoreMesh(
        core_axis_name="c", subcore_axis_name="s", num_cores=2, num_subcores=16
    )

    @functools.partial(
        pl.kernel,
        out_type=jax.ShapeDtypeStruct((TOK, D), jnp.float32),
        mesh=mesh,
        scratch_types=[
            pltpu.VMEM((C,), jnp.int32),
            pltpu.VMEM((C,), jnp.int32),
            pltpu.VMEM((C, D), jnp.float32),
            pltpu.VMEM((C, D), jnp.float32),
            pltpu.VMEM((C, D), jnp.float32),
            pltpu.SemaphoreType.DMA,
            pltpu.SemaphoreType.DMA,
        ],
        interpret=interpret,
    )
    def emb_kernel(x_hbm, seg_hbm, tok_tab, seg_tab, pe_hbm, out_hbm,
                   idx_v, sidx_v, acc_v, pe_v, seg_v, sem0, sem1):
        wid = lax.axis_index("s") * 2 + lax.axis_index("c")
        row0 = wid * rows_per_w

        def lc_body(lc, carry):
            l0 = lc * C
            pltpu.sync_copy(pe_hbm.at[pl.ds(l0, C)], pe_v)

            def b_body(bi, carry2):
                base = (row0 + bi) * L + l0
                pltpu.sync_copy(x_hbm.at[pl.ds(base, C)], idx_v)
                pltpu.sync_copy(seg_hbm.at[pl.ds(base, C)], sidx_v)
                cp_t = pltpu.async_copy(tok_tab.at[idx_v], acc_v, sem0)
                cp_s = pltpu.async_copy(seg_tab.at[sidx_v], seg_v, sem1)
                cp_t.wait()
                cp_s.wait()

                def add_row(i, carry3):
                    for c in range(NSLICE):
                        sl = pl.ds(c * NLANE, NLANE)
                        acc_v[i, sl] = acc_v[i, sl] + pe_v[i, sl] + seg_v[i, sl]
                    return carry3

                lax.fori_loop(0, C, add_row, None)
                pltpu.sync_copy(acc_v, out_hbm.at[pl.ds(base, C)])
                return carry2

            lax.fori_loop(0, rows_per_w, b_body, None)
            return carry

        lax.fori_loop(0, n_lc, lc_body, None)

    return emb_kernel


def kernel(x, segment_label, token_table, segment_table):
    B, L = x.shape
    x_flat = x.reshape(-1).astype(jnp.int32)
    s_flat = segment_label.reshape(-1).astype(jnp.int32)
    pe = jnp.asarray(_pe_table()[:L])
    out = _make_kernel(B, L)(x_flat, s_flat, token_table, segment_table, pe)
    return out.reshape(B, L, D)
